# Initial kernel scaffold; baseline (speedup 1.0000x reference)
import jax
import jax.numpy as jnp
from jax.experimental import pallas as pl


def kernel(V, E, edges, W_f, W_a, b_a):
    raise NotImplementedError("write your pallas kernel here")


# speedup vs baseline: 2185.7532x; 2185.7532x over previous
import functools

import jax
import jax.numpy as jnp
from jax import lax
from jax.experimental import pallas as pl
from jax.experimental.pallas import tpu as pltpu
from jax.experimental.pallas import tpu_sc as plsc

NN = 10000
NE = 160000
DF = 128
NW = 32
EP = 160256
CH = EP // NW
ITERS = CH // 16

_P = jax.lax.Precision.HIGHEST


def _k1_body(v_ref, wf_ref, wa_ref, h_ref, p_ref, q_ref):
    v = v_ref[...]
    h = lax.dot_general(v, wf_ref[...], (((1,), (1,)), ((), ())),
                        precision=_P, preferred_element_type=jnp.float32)
    h_ref[...] = h
    a1 = wa_ref[:, 0:128]
    a2 = wa_ref[:, 128:256]
    p_ref[...] = jnp.sum(h * a1, axis=1, keepdims=True)
    q_ref[...] = jnp.sum(h * a2, axis=1, keepdims=True)


def _k2_body(e_ref, wa_ref, b_ref, le_ref):
    a3 = wa_ref[:, 256:384]
    le_ref[...] = jnp.sum(e_ref[...] * a3, axis=1, keepdims=True) + b_ref[0, 0]


def _k5_body(h_ref, dp_ref, o_ref):
    d = jnp.sum(dp_ref[...], axis=0)
    o_ref[...] = h_ref[...] * (d / (d + 1e-8))[:, None]


_sc_mesh = plsc.VectorSubcoreMesh(core_axis_name="c", subcore_axis_name="s")
_sc_params = pltpu.CompilerParams(needs_layout_passes=False)


@functools.partial(
    pl.kernel, mesh=_sc_mesh, compiler_params=_sc_params,
    out_type=[jax.ShapeDtypeStruct((EP,), jnp.float32),
              jax.ShapeDtypeStruct((NW * 16,), jnp.float32)],
    scratch_types=[pltpu.VMEM((NN,), jnp.float32),
                   pltpu.VMEM((NN,), jnp.float32),
                   pltpu.VMEM((CH,), jnp.int32),
                   pltpu.VMEM((CH,), jnp.int32),
                   pltpu.VMEM((CH,), jnp.float32),
                   pltpu.VMEM((CH,), jnp.float32),
                   pltpu.VMEM((16,), jnp.float32)])
def _k3(p_hbm, q_hbm, s_hbm, r_hbm, le_hbm, lo_hbm, pmax_hbm,
        p_v, q_v, s_v, r_v, le_v, lo_v, mx_v):
    wid = lax.axis_index("s") * 2 + lax.axis_index("c")
    base = wid * CH
    pltpu.sync_copy(p_hbm, p_v)
    pltpu.sync_copy(q_hbm, q_v)
    pltpu.sync_copy(s_hbm.at[pl.ds(base, CH)], s_v)
    pltpu.sync_copy(r_hbm.at[pl.ds(base, CH)], r_v)
    pltpu.sync_copy(le_hbm.at[pl.ds(base, CH)], le_v)

    def body(i, m16):
        sl = pl.ds(i * 16, 16)
        lg = (plsc.load_gather(p_v, [s_v[sl]])
              + plsc.load_gather(q_v, [r_v[sl]])
              + le_v[sl])
        lg = jnp.where(lg >= 0.0, lg, 0.2 * lg)
        lo_v[sl] = lg
        return jnp.maximum(m16, lg)

    m16 = lax.fori_loop(0, ITERS, body, jnp.full((16,), -3e38, jnp.float32))
    mx_v[...] = m16
    pltpu.sync_copy(lo_v, lo_hbm.at[pl.ds(base, CH)])
    pltpu.sync_copy(mx_v, pmax_hbm.at[pl.ds(wid * 16, 16)])


@functools.partial(
    pl.kernel, mesh=_sc_mesh, compiler_params=_sc_params,
    out_type=jax.ShapeDtypeStruct((NW, NN), jnp.float32),
    scratch_types=[pltpu.VMEM((CH,), jnp.float32),
                   pltpu.VMEM((CH,), jnp.int32),
                   pltpu.VMEM((NN,), jnp.float32),
                   pltpu.VMEM((NW * 16,), jnp.float32)])
def _k4(lo_hbm, s_hbm, pmax_hbm, dpart_hbm, lo_v, s_v, d_v, pm_v):
    wid = lax.axis_index("s") * 2 + lax.axis_index("c")
    base = wid * CH
    pltpu.sync_copy(lo_hbm.at[pl.ds(base, CH)], lo_v)
    pltpu.sync_copy(s_hbm.at[pl.ds(base, CH)], s_v)
    pltpu.sync_copy(pmax_hbm, pm_v)

    m16 = lax.fori_loop(
        0, NW,
        lambda j, m: jnp.maximum(m, pm_v[pl.ds(j * 16, 16)]),
        jnp.full((16,), -3e38, jnp.float32))
    ms = jnp.full((16,), jnp.max(m16))

    def zero(j, c):
        d_v[pl.ds(j * 16, 16)] = jnp.zeros((16,), jnp.float32)
        return c
    lax.fori_loop(0, NN // 16, zero, 0)

    def body(i, c):
        sl = pl.ds(i * 16, 16)
        att = jnp.exp(lo_v[sl] - ms)
        plsc.addupdate_scatter(d_v, [s_v[sl]], att)
        return c
    lax.fori_loop(0, ITERS, body, 0)
    pltpu.sync_copy(d_v, dpart_hbm.at[wid])


def kernel(V, E, edges, W_f, W_a, b_a):
    V2 = V[0]
    E2 = E[0]
    pad = EP - NE
    sp = jnp.concatenate([edges[0, :, 0], jnp.zeros((pad,), jnp.int32)])
    rp = jnp.concatenate([edges[0, :, 1], jnp.zeros((pad,), jnp.int32)])

    H, p2, q2 = pl.pallas_call(
        _k1_body,
        grid=(10,),
        in_specs=[pl.BlockSpec((1000, 128), lambda i: (i, 0)),
                  pl.BlockSpec((128, 128), lambda i: (0, 0)),
                  pl.BlockSpec((1, 384), lambda i: (0, 0))],
        out_specs=[pl.BlockSpec((1000, 128), lambda i: (i, 0)),
                   pl.BlockSpec((1000, 1), lambda i: (i, 0)),
                   pl.BlockSpec((1000, 1), lambda i: (i, 0))],
        out_shape=[jax.ShapeDtypeStruct((NN, 128), jnp.float32),
                   jax.ShapeDtypeStruct((NN, 1), jnp.float32),
                   jax.ShapeDtypeStruct((NN, 1), jnp.float32)],
    )(V2, W_f, W_a)
    p = p2.reshape(NN)
    q = q2.reshape(NN)

    le = pl.pallas_call(
        _k2_body,
        grid=(80,),
        in_specs=[pl.BlockSpec((2000, 128), lambda i: (i, 0)),
                  pl.BlockSpec((1, 384), lambda i: (0, 0)),
                  pl.BlockSpec((1, 1), lambda i: (0, 0))],
        out_specs=pl.BlockSpec((2000, 1), lambda i: (i, 0)),
        out_shape=jax.ShapeDtypeStruct((NE, 1), jnp.float32),
    )(E2, W_a, b_a.reshape(1, 1)).reshape(NE)
    lep = jnp.concatenate([le, jnp.full((pad,), -1e30, jnp.float32)])

    lo, pmax = _k3(p, q, sp, rp, lep)
    dpart = _k4(lo, sp, pmax)

    h = pl.pallas_call(
        _k5_body,
        out_shape=jax.ShapeDtypeStruct((NN, 128), jnp.float32),
    )(H, dpart)
    return h.reshape(1, NN, DF)


# speedup vs baseline: 2224.3794x; 1.0177x over previous
import functools

import jax
import jax.numpy as jnp
from jax import lax
from jax.experimental import pallas as pl
from jax.experimental.pallas import tpu as pltpu
from jax.experimental.pallas import tpu_sc as plsc

NN = 10000
NE = 160000
DF = 128
NW = 32
EP = 160256
CH = EP // NW
ITERS = CH // 16

_P = jax.lax.Precision.HIGHEST


def _k1_body(v_ref, wf_ref, wa_ref, p_ref, q_ref):
    h = lax.dot_general(v_ref[...], wf_ref[...], (((1,), (1,)), ((), ())),
                        precision=_P, preferred_element_type=jnp.float32)
    p_ref[...] = jnp.sum(h * wa_ref[:, 0:128], axis=1, keepdims=True)
    q_ref[...] = jnp.sum(h * wa_ref[:, 128:256], axis=1, keepdims=True)


def _k2_body(e_ref, wa_ref, b_ref, le_ref):
    a3 = wa_ref[:, 256:384]
    le_ref[...] = jnp.sum(e_ref[...] * a3, axis=1, keepdims=True) + b_ref[0, 0]


def _k4_body(v_ref, wf_ref, dp_ref, mv_ref, o_ref):
    mv = mv_ref[:, 0:1]
    scale = jnp.exp(mv - jnp.max(mv))
    d = jnp.sum(dp_ref[...] * scale, axis=0)
    h = lax.dot_general(v_ref[...], wf_ref[...], (((1,), (1,)), ((), ())),
                        precision=_P, preferred_element_type=jnp.float32)
    o_ref[...] = h * (d / (d + 1e-8))[:, None]


_sc_mesh = plsc.VectorSubcoreMesh(core_axis_name="c", subcore_axis_name="s")
_sc_params = pltpu.CompilerParams(needs_layout_passes=False)


@functools.partial(
    pl.kernel, mesh=_sc_mesh, compiler_params=_sc_params,
    out_type=[jax.ShapeDtypeStruct((NW, NN), jnp.float32),
              jax.ShapeDtypeStruct((NW, 16), jnp.float32)],
    scratch_types=[pltpu.VMEM((NN,), jnp.float32),
                   pltpu.VMEM((NN,), jnp.float32),
                   pltpu.VMEM((CH,), jnp.int32),
                   pltpu.VMEM((CH,), jnp.int32),
                   pltpu.VMEM((CH,), jnp.float32),
                   pltpu.VMEM((CH,), jnp.float32),
                   pltpu.VMEM((NN,), jnp.float32),
                   pltpu.VMEM((16,), jnp.float32)])
def _k3(p_hbm, q_hbm, s_hbm, r_hbm, le_hbm, dpart_hbm, mvec_hbm,
        p_v, q_v, s_v, r_v, le_v, lo_v, d_v, mx_v):
    wid = lax.axis_index("s") * 2 + lax.axis_index("c")
    base = wid * CH
    pltpu.sync_copy(p_hbm, p_v)
    pltpu.sync_copy(q_hbm, q_v)
    pltpu.sync_copy(s_hbm.at[pl.ds(base, CH)], s_v)
    pltpu.sync_copy(r_hbm.at[pl.ds(base, CH)], r_v)
    pltpu.sync_copy(le_hbm.at[pl.ds(base, CH)], le_v)

    def logit_body(i, m16):
        sl = pl.ds(i * 16, 16)
        lg = (plsc.load_gather(p_v, [s_v[sl]])
              + plsc.load_gather(q_v, [r_v[sl]])
              + le_v[sl])
        lg = jnp.where(lg >= 0.0, lg, 0.2 * lg)
        lo_v[sl] = lg
        return jnp.maximum(m16, lg)

    m16 = lax.fori_loop(0, ITERS, logit_body,
                        jnp.full((16,), -3e38, jnp.float32))
    ms = jnp.full((16,), jnp.max(m16))
    mx_v[...] = ms

    def zero_body(j, c):
        d_v[pl.ds(j * 16, 16)] = jnp.zeros((16,), jnp.float32)
        return c
    lax.fori_loop(0, NN // 16, zero_body, 0)

    def acc_body(i, c):
        sl = pl.ds(i * 16, 16)
        att = jnp.exp(lo_v[sl] - ms)
        plsc.addupdate_scatter(d_v, [s_v[sl]], att)
        return c
    lax.fori_loop(0, ITERS, acc_body, 0)

    pltpu.sync_copy(d_v, dpart_hbm.at[wid])
    pltpu.sync_copy(mx_v, mvec_hbm.at[wid])


def kernel(V, E, edges, W_f, W_a, b_a):
    V2 = V[0]
    E2 = E[0]
    pad = EP - NE
    sp = jnp.concatenate([edges[0, :, 0], jnp.zeros((pad,), jnp.int32)])
    rp = jnp.concatenate([edges[0, :, 1], jnp.zeros((pad,), jnp.int32)])

    p2, q2 = pl.pallas_call(
        _k1_body,
        grid=(10,),
        in_specs=[pl.BlockSpec((1000, 128), lambda i: (i, 0)),
                  pl.BlockSpec((128, 128), lambda i: (0, 0)),
                  pl.BlockSpec((1, 384), lambda i: (0, 0))],
        out_specs=[pl.BlockSpec((1000, 1), lambda i: (i, 0)),
                   pl.BlockSpec((1000, 1), lambda i: (i, 0))],
        out_shape=[jax.ShapeDtypeStruct((NN, 1), jnp.float32),
                   jax.ShapeDtypeStruct((NN, 1), jnp.float32)],
    )(V2, W_f, W_a)
    p = p2.reshape(NN)
    q = q2.reshape(NN)

    le = pl.pallas_call(
        _k2_body,
        grid=(80,),
        in_specs=[pl.BlockSpec((2000, 128), lambda i: (i, 0)),
                  pl.BlockSpec((1, 384), lambda i: (0, 0)),
                  pl.BlockSpec((1, 1), lambda i: (0, 0))],
        out_specs=pl.BlockSpec((2000, 1), lambda i: (i, 0)),
        out_shape=jax.ShapeDtypeStruct((NE, 1), jnp.float32),
    )(E2, W_a, b_a.reshape(1, 1)).reshape(NE)
    lep = jnp.concatenate([le, jnp.full((pad,), -1e30, jnp.float32)])

    dpart, mvec = _k3(p, q, sp, rp, lep)

    h = pl.pallas_call(
        _k4_body,
        out_shape=jax.ShapeDtypeStruct((NN, 128), jnp.float32),
    )(V2, W_f, dpart, mvec)
    return h.reshape(1, NN, DF)


# speedup vs baseline: 2828.0165x; 1.2714x over previous
import functools

import jax
import jax.numpy as jnp
from jax import lax
from jax.experimental import pallas as pl
from jax.experimental.pallas import tpu as pltpu
from jax.experimental.pallas import tpu_sc as plsc

NN = 10000
NE = 160000
DF = 128
NW = 32
CH = NE // NW
FULL = CH // 16
TAIL = CH - FULL * 16
CHP = (FULL + 1) * 16

_P = jax.lax.Precision.HIGHEST


def _k1_body(v_ref, wf_ref, wa_ref, p_ref, q_ref):
    h = lax.dot_general(v_ref[...], wf_ref[...], (((1,), (1,)), ((), ())),
                        precision=_P, preferred_element_type=jnp.float32)
    p_ref[...] = jnp.sum(h * wa_ref[:, 0:128], axis=1)
    q_ref[...] = jnp.sum(h * wa_ref[:, 128:256], axis=1)


def _k2_body(e_ref, wa_ref, b_ref, le_ref):
    a3 = wa_ref[:, 256:384]
    le_ref[...] = jnp.sum(e_ref[...] * a3, axis=1) + b_ref[0, 0]


def _k4_body(v_ref, wf_ref, dp_ref, mv_ref, o_ref):
    mv = mv_ref[:, 0:1]
    scale = jnp.exp(mv - jnp.max(mv))
    d = jnp.sum(dp_ref[...] * scale, axis=0)
    h = lax.dot_general(v_ref[...], wf_ref[...], (((1,), (1,)), ((), ())),
                        precision=_P, preferred_element_type=jnp.float32)
    o_ref[...] = h * (d / (d + 1e-8))[:, None]


_sc_mesh = plsc.VectorSubcoreMesh(core_axis_name="c", subcore_axis_name="s")
_sc_params = pltpu.CompilerParams(needs_layout_passes=False)


@functools.partial(
    pl.kernel, mesh=_sc_mesh, compiler_params=_sc_params,
    out_type=[jax.ShapeDtypeStruct((NW, NN), jnp.float32),
              jax.ShapeDtypeStruct((NW, 16), jnp.float32)],
    scratch_types=[pltpu.VMEM((NN,), jnp.float32),
                   pltpu.VMEM((NN,), jnp.float32),
                   pltpu.VMEM((CHP,), jnp.int32),
                   pltpu.VMEM((CHP,), jnp.int32),
                   pltpu.VMEM((CHP,), jnp.float32),
                   pltpu.VMEM((CHP,), jnp.float32),
                   pltpu.VMEM((NN,), jnp.float32),
                   pltpu.VMEM((16,), jnp.float32)])
def _k3(p_hbm, q_hbm, s_hbm, r_hbm, le_hbm, dpart_hbm, mvec_hbm,
        p_v, q_v, s_v, r_v, le_v, lo_v, d_v, mx_v):
    wid = lax.axis_index("s") * 2 + lax.axis_index("c")
    base = wid * CH
    pltpu.sync_copy(p_hbm, p_v)
    pltpu.sync_copy(q_hbm, q_v)
    pltpu.sync_copy(s_hbm.at[pl.ds(base, CH)], s_v.at[pl.ds(0, CH)])
    pltpu.sync_copy(r_hbm.at[pl.ds(base, CH)], r_v.at[pl.ds(0, CH)])
    pltpu.sync_copy(le_hbm.at[pl.ds(base, CH)], le_v.at[pl.ds(0, CH)])

    iota = lax.iota(jnp.int32, 16)

    def logit_body(i, m16):
        sl = pl.ds(i * 16, 16)
        lg = (plsc.load_gather(p_v, [s_v[sl]])
              + plsc.load_gather(q_v, [r_v[sl]])
              + le_v[sl])
        lg = jnp.where(lg >= 0.0, lg, 0.2 * lg)
        lo_v[sl] = lg
        return jnp.maximum(m16, lg)

    m16 = lax.fori_loop(0, FULL, logit_body,
                        jnp.full((16,), -3e38, jnp.float32))

    tmask = iota < TAIL
    sl = pl.ds(FULL * 16, 16)
    s16 = jnp.where(tmask, s_v[sl], 0)
    r16 = jnp.where(tmask, r_v[sl], 0)
    lg = (plsc.load_gather(p_v, [s16], mask=tmask)
          + plsc.load_gather(q_v, [r16], mask=tmask)
          + jnp.where(tmask, le_v[sl], 0.0))
    lg = jnp.where(lg >= 0.0, lg, 0.2 * lg)
    lg = jnp.where(tmask, lg, -3e38)
    s_v[sl] = s16
    lo_v[sl] = lg
    m16 = jnp.maximum(m16, lg)

    ms = jnp.full((16,), jnp.max(m16))
    mx_v[...] = ms

    def zero_body(j, c):
        d_v[pl.ds(j * 16, 16)] = jnp.zeros((16,), jnp.float32)
        return c
    lax.fori_loop(0, NN // 16, zero_body, 0)

    def acc_body(i, c):
        sl = pl.ds(i * 16, 16)
        att = jnp.exp(lo_v[sl] - ms)
        plsc.addupdate_scatter(d_v, [s_v[sl]], att)
        return c
    lax.fori_loop(0, FULL + 1, acc_body, 0)

    pltpu.sync_copy(d_v, dpart_hbm.at[wid])
    pltpu.sync_copy(mx_v, mvec_hbm.at[wid])


def kernel(V, E, edges, W_f, W_a, b_a):
    V2 = V[0]
    E2 = E[0]

    p, q = pl.pallas_call(
        _k1_body,
        out_shape=[jax.ShapeDtypeStruct((NN,), jnp.float32),
                   jax.ShapeDtypeStruct((NN,), jnp.float32)],
    )(V2, W_f, W_a)

    le = pl.pallas_call(
        _k2_body,
        grid=(20,),
        in_specs=[pl.BlockSpec((8192, 128), lambda i: (i, 0)),
                  pl.BlockSpec((1, 384), lambda i: (0, 0)),
                  pl.BlockSpec((1, 1), lambda i: (0, 0))],
        out_specs=pl.BlockSpec((8192,), lambda i: (i,)),
        out_shape=jax.ShapeDtypeStruct((NE,), jnp.float32),
    )(E2, W_a, b_a.reshape(1, 1))

    dpart, mvec = _k3(p, q, edges[0, :, 0], edges[0, :, 1], le)

    h = pl.pallas_call(
        _k4_body,
        out_shape=jax.ShapeDtypeStruct((NN, 128), jnp.float32),
    )(V2, W_f, dpart, mvec)
    return h.reshape(1, NN, DF)


# speedup vs baseline: 2954.9319x; 1.0449x over previous
import functools

import jax
import jax.numpy as jnp
from jax import lax
from jax.experimental import pallas as pl
from jax.experimental.pallas import tpu as pltpu
from jax.experimental.pallas import tpu_sc as plsc

NN = 10000
NE = 160000
DF = 128
NW = 32
CH = NE // NW
FULL = CH // 16
TAIL = CH - FULL * 16
CHP = (FULL + 1) * 16

_P = jax.lax.Precision.HIGHEST


def _k1_body(v_ref, wf_ref, wa_ref, p_ref, q_ref):
    h = lax.dot_general(v_ref[...], wf_ref[...], (((1,), (1,)), ((), ())),
                        precision=_P, preferred_element_type=jnp.float32)
    p_ref[...] = jnp.sum(h * wa_ref[:, 0:128], axis=1)
    q_ref[...] = jnp.sum(h * wa_ref[:, 128:256], axis=1)


def _k2_body(e_ref, wa_ref, b_ref, le_ref):
    a3 = wa_ref[:, 256:384]
    le_ref[...] = jnp.sum(e_ref[...] * a3, axis=1) + b_ref[0, 0]


def _k4_body(v_ref, wf_ref, dp_ref, mv_ref, o_ref):
    mv = mv_ref[:, 0:1]
    scale = jnp.exp(mv - jnp.max(mv))
    d = jnp.sum(dp_ref[...] * scale, axis=0)
    h = lax.dot_general(v_ref[...], wf_ref[...], (((1,), (1,)), ((), ())),
                        precision=_P, preferred_element_type=jnp.float32)
    o_ref[...] = h * (d / (d + 1e-8))[:, None]


_sc_mesh = plsc.VectorSubcoreMesh(core_axis_name="c", subcore_axis_name="s")
_sc_params = pltpu.CompilerParams(needs_layout_passes=False)


@functools.partial(
    pl.kernel, mesh=_sc_mesh, compiler_params=_sc_params,
    out_type=[jax.ShapeDtypeStruct((NW, NN), jnp.float32),
              jax.ShapeDtypeStruct((NW, 16), jnp.float32)],
    scratch_types=[pltpu.VMEM((NN,), jnp.float32),
                   pltpu.VMEM((NN,), jnp.float32),
                   pltpu.VMEM((CHP,), jnp.int32),
                   pltpu.VMEM((CHP,), jnp.int32),
                   pltpu.VMEM((CHP,), jnp.float32),
                   pltpu.VMEM((CHP,), jnp.float32),
                   pltpu.VMEM((NN,), jnp.float32),
                   pltpu.VMEM((16,), jnp.float32)])
def _k3(p_hbm, q_hbm, s_hbm, r_hbm, le_hbm, dpart_hbm, mvec_hbm,
        p_v, q_v, s_v, r_v, le_v, lo_v, d_v, mx_v):
    wid = lax.axis_index("s") * 2 + lax.axis_index("c")
    base = wid * CH
    pltpu.sync_copy(p_hbm, p_v)
    pltpu.sync_copy(q_hbm, q_v)
    pltpu.sync_copy(s_hbm.at[pl.ds(base, CH)], s_v.at[pl.ds(0, CH)])
    pltpu.sync_copy(r_hbm.at[pl.ds(base, CH)], r_v.at[pl.ds(0, CH)])
    pltpu.sync_copy(le_hbm.at[pl.ds(base, CH)], le_v.at[pl.ds(0, CH)])

    iota = lax.iota(jnp.int32, 16)

    def logit_body(i, m16):
        sl = pl.ds(i * 16, 16)
        lg = (plsc.load_gather(p_v, [s_v[sl]])
              + plsc.load_gather(q_v, [r_v[sl]])
              + le_v[sl])
        lg = jnp.where(lg >= 0.0, lg, 0.2 * lg)
        lo_v[sl] = lg
        return jnp.maximum(m16, lg)

    m16 = lax.fori_loop(0, FULL, logit_body,
                        jnp.full((16,), -3e38, jnp.float32))

    tmask = iota < TAIL
    sl = pl.ds(FULL * 16, 16)
    s16 = jnp.where(tmask, s_v[sl], 0)
    r16 = jnp.where(tmask, r_v[sl], 0)
    lg = (plsc.load_gather(p_v, [s16], mask=tmask)
          + plsc.load_gather(q_v, [r16], mask=tmask)
          + jnp.where(tmask, le_v[sl], 0.0))
    lg = jnp.where(lg >= 0.0, lg, 0.2 * lg)
    lg = jnp.where(tmask, lg, -3e38)
    s_v[sl] = s16
    lo_v[sl] = lg
    m16 = jnp.maximum(m16, lg)

    ms = jnp.full((16,), jnp.max(m16))
    mx_v[...] = ms

    def zero_body(j, c):
        d_v[pl.ds(j * 16, 16)] = jnp.zeros((16,), jnp.float32)
        return c
    lax.fori_loop(0, NN // 16, zero_body, 0)

    def acc_body(i, c):
        sl = pl.ds(i * 16, 16)
        att = jnp.exp(lo_v[sl] - ms)
        plsc.addupdate_scatter(d_v, [s_v[sl]], att)
        return c
    lax.fori_loop(0, FULL + 1, acc_body, 0)

    pltpu.sync_copy(d_v, dpart_hbm.at[wid])
    pltpu.sync_copy(mx_v, mvec_hbm.at[wid])


def kernel(V, E, edges, W_f, W_a, b_a):
    V2 = V[0]
    E2 = E[0]

    p, q = pl.pallas_call(
        _k1_body,
        grid=(10,),
        in_specs=[pl.BlockSpec((1024, 128), lambda i: (i, 0)),
                  pl.BlockSpec((128, 128), lambda i: (0, 0)),
                  pl.BlockSpec((1, 384), lambda i: (0, 0))],
        out_specs=[pl.BlockSpec((1024,), lambda i: (i,)),
                   pl.BlockSpec((1024,), lambda i: (i,))],
        out_shape=[jax.ShapeDtypeStruct((NN,), jnp.float32),
                   jax.ShapeDtypeStruct((NN,), jnp.float32)],
    )(V2, W_f, W_a)

    le = pl.pallas_call(
        _k2_body,
        grid=(10,),
        in_specs=[pl.BlockSpec((16384, 128), lambda i: (i, 0)),
                  pl.BlockSpec((1, 384), lambda i: (0, 0)),
                  pl.BlockSpec((1, 1), lambda i: (0, 0))],
        out_specs=pl.BlockSpec((16384,), lambda i: (i,)),
        out_shape=jax.ShapeDtypeStruct((NE,), jnp.float32),
    )(E2, W_a, b_a.reshape(1, 1))

    dpart, mvec = _k3(p, q, edges[0, :, 0], edges[0, :, 1], le)

    h = pl.pallas_call(
        _k4_body,
        out_shape=jax.ShapeDtypeStruct((NN, 128), jnp.float32),
    )(V2, W_f, dpart, mvec)
    return h.reshape(1, NN, DF)


# speedup vs baseline: 3127.4381x; 1.0584x over previous
import functools

import jax
import jax.numpy as jnp
from jax import lax
from jax.experimental import pallas as pl
from jax.experimental.pallas import tpu as pltpu
from jax.experimental.pallas import tpu_sc as plsc

NN = 10000
NE = 160000
DF = 128
NW = 32
CH = NE // NW
FULL = CH // 16
TAIL = CH - FULL * 16
CHP = (FULL + 1) * 16

_P = jax.lax.Precision.HIGHEST


def _k2_body(e_ref, v_ref, wf_ref, wa_ref, b_ref, le_ref, p_ref, q_ref):
    a3 = wa_ref[:, 256:384]
    le_ref[...] = jnp.sum(e_ref[...] * a3, axis=1) + b_ref[0, 0]
    h = lax.dot_general(v_ref[...], wf_ref[...], (((1,), (1,)), ((), ())),
                        precision=_P, preferred_element_type=jnp.float32)
    p_ref[...] = jnp.sum(h * wa_ref[:, 0:128], axis=1)
    q_ref[...] = jnp.sum(h * wa_ref[:, 128:256], axis=1)


def _k4_body(v_ref, wf_ref, dp_ref, mv_ref, o_ref):
    mv = mv_ref[:, 0:1]
    scale = jnp.exp(mv - jnp.max(mv))
    d = jnp.sum(dp_ref[...] * scale, axis=0)
    h = lax.dot_general(v_ref[...], wf_ref[...], (((1,), (1,)), ((), ())),
                        precision=_P, preferred_element_type=jnp.float32)
    o_ref[...] = h * (d / (d + 1e-8))[:, None]


_sc_mesh = plsc.VectorSubcoreMesh(core_axis_name="c", subcore_axis_name="s")
_sc_params = pltpu.CompilerParams(needs_layout_passes=False)


@functools.partial(
    pl.kernel, mesh=_sc_mesh, compiler_params=_sc_params,
    out_type=[jax.ShapeDtypeStruct((NW, NN), jnp.float32),
              jax.ShapeDtypeStruct((NW, 16), jnp.float32)],
    scratch_types=[pltpu.VMEM((NN,), jnp.float32),
                   pltpu.VMEM((NN,), jnp.float32),
                   pltpu.VMEM((CHP,), jnp.int32),
                   pltpu.VMEM((CHP,), jnp.int32),
                   pltpu.VMEM((CHP,), jnp.float32),
                   pltpu.VMEM((CHP,), jnp.float32),
                   pltpu.VMEM((NN,), jnp.float32),
                   pltpu.VMEM((16,), jnp.float32)])
def _k3(p_hbm, q_hbm, s_hbm, r_hbm, le_hbm, dpart_hbm, mvec_hbm,
        p_v, q_v, s_v, r_v, le_v, lo_v, d_v, mx_v):
    wid = lax.axis_index("s") * 2 + lax.axis_index("c")
    base = wid * CH
    pltpu.sync_copy(p_hbm, p_v)
    pltpu.sync_copy(q_hbm, q_v)
    pltpu.sync_copy(s_hbm.at[pl.ds(base, CH)], s_v.at[pl.ds(0, CH)])
    pltpu.sync_copy(r_hbm.at[pl.ds(base, CH)], r_v.at[pl.ds(0, CH)])
    pltpu.sync_copy(le_hbm.at[pl.ds(base, CH)], le_v.at[pl.ds(0, CH)])

    iota = lax.iota(jnp.int32, 16)

    def logit_body(i, m16):
        sl = pl.ds(i * 16, 16)
        lg = (plsc.load_gather(p_v, [s_v[sl]])
              + plsc.load_gather(q_v, [r_v[sl]])
              + le_v[sl])
        lg = jnp.where(lg >= 0.0, lg, 0.2 * lg)
        lo_v[sl] = lg
        return jnp.maximum(m16, lg)

    m16 = lax.fori_loop(0, FULL, logit_body,
                        jnp.full((16,), -3e38, jnp.float32))

    tmask = iota < TAIL
    sl = pl.ds(FULL * 16, 16)
    s16 = jnp.where(tmask, s_v[sl], 0)
    r16 = jnp.where(tmask, r_v[sl], 0)
    lg = (plsc.load_gather(p_v, [s16], mask=tmask)
          + plsc.load_gather(q_v, [r16], mask=tmask)
          + jnp.where(tmask, le_v[sl], 0.0))
    lg = jnp.where(lg >= 0.0, lg, 0.2 * lg)
    lg = jnp.where(tmask, lg, -3e38)
    s_v[sl] = s16
    lo_v[sl] = lg
    m16 = jnp.maximum(m16, lg)

    ms = jnp.full((16,), jnp.max(m16))
    mx_v[...] = ms

    def zero_body(j, c):
        d_v[pl.ds(j * 16, 16)] = jnp.zeros((16,), jnp.float32)
        return c
    lax.fori_loop(0, NN // 16, zero_body, 0)

    def acc_body(i, c):
        sl = pl.ds(i * 16, 16)
        att = jnp.exp(lo_v[sl] - ms)
        plsc.addupdate_scatter(d_v, [s_v[sl]], att)
        return c
    lax.fori_loop(0, FULL + 1, acc_body, 0)

    pltpu.sync_copy(d_v, dpart_hbm.at[wid])
    pltpu.sync_copy(mx_v, mvec_hbm.at[wid])


def kernel(V, E, edges, W_f, W_a, b_a):
    V2 = V[0]
    E2 = E[0]

    le, p, q = pl.pallas_call(
        _k2_body,
        grid=(10,),
        in_specs=[pl.BlockSpec((16384, 128), lambda i: (i, 0)),
                  pl.BlockSpec((1024, 128), lambda i: (i, 0)),
                  pl.BlockSpec((128, 128), lambda i: (0, 0)),
                  pl.BlockSpec((1, 384), lambda i: (0, 0)),
                  pl.BlockSpec((1, 1), lambda i: (0, 0))],
        out_specs=[pl.BlockSpec((16384,), lambda i: (i,)),
                   pl.BlockSpec((1024,), lambda i: (i,)),
                   pl.BlockSpec((1024,), lambda i: (i,))],
        out_shape=[jax.ShapeDtypeStruct((NE,), jnp.float32),
                   jax.ShapeDtypeStruct((NN,), jnp.float32),
                   jax.ShapeDtypeStruct((NN,), jnp.float32)],
    )(E2, V2, W_f, W_a, b_a.reshape(1, 1))

    dpart, mvec = _k3(p, q, edges[0, :, 0], edges[0, :, 1], le)

    h = pl.pallas_call(
        _k4_body,
        out_shape=jax.ShapeDtypeStruct((NN, 128), jnp.float32),
    )(V2, W_f, dpart, mvec)
    return h.reshape(1, NN, DF)
